# baseline (speedup 1.0000x reference)
import jax
import jax.numpy as jnp
from jax import lax
from jax.experimental import pallas as pl
from jax.experimental.pallas import tpu as pltpu
from jax.experimental.pallas import tpu_sc as plsc

KMER_SIZE = 4
BATCH = 1024
SEQ_LEN = 512
EMBED_DIM = 64
PAD_DIM = 128

NUM_CORES = 2
NUM_SUBCORES = 16
NUM_WORKERS = NUM_CORES * NUM_SUBCORES
PER_WORKER = BATCH * SEQ_LEN // NUM_WORKERS
CHUNK = 256
N_CHUNKS = PER_WORKER // CHUNK
PROWS = CHUNK // 2
OUT_ROWS = BATCH * SEQ_LEN * EMBED_DIM // 128


def _sc_body(idx_hbm, table_hbm, out_hbm,
             i0, i1, g_v, p_v, is0, is1, gs0, gs1, ss0, ss1):
    ibufs = (i0, i1)
    isems = (is0, is1)
    gsems = (gs0, gs1)
    ssems = (ss0, ss1)
    wid = lax.axis_index("s") * NUM_CORES + lax.axis_index("c")
    idx_base = wid * PER_WORKER
    out_base = wid * N_CHUNKS * PROWS

    def fire_idx(c, slot):
        pltpu.async_copy(
            idx_hbm.at[pl.ds(idx_base + c * CHUNK, CHUNK)], ibufs[slot],
            isems[slot],
        )

    def wait_idx(slot):
        pltpu.make_async_copy(
            idx_hbm.at[pl.ds(idx_base, CHUNK)], ibufs[slot], isems[slot]
        ).wait()

    def fire_gather(c, slot):
        pltpu.async_copy(table_hbm.at[ibufs[slot]], g_v.at[slot], gsems[slot])

    def wait_gather(slot):
        pltpu.make_async_copy(
            table_hbm.at[ibufs[slot]], g_v.at[slot], gsems[slot]
        ).wait()

    def fire_scatter(c, slot):
        pltpu.async_copy(
            p_v.at[slot], out_hbm.at[pl.ds(out_base + c * PROWS, PROWS)],
            ssems[slot],
        )

    def wait_scatter(slot):
        pltpu.make_async_copy(
            p_v.at[slot], out_hbm.at[pl.ds(0, PROWS)], ssems[slot]
        ).wait()

    def repack(slot):
        def rows4(i, _):
            r0 = i * 4
            for dr in range(4):
                r = r0 + dr
                for l in range(EMBED_DIM // 16):
                    p_v[slot, r, pl.ds(l * 16, 16)] = g_v[
                        slot, 2 * r, pl.ds(l * 16, 16)
                    ]
                    p_v[slot, r, pl.ds(EMBED_DIM + l * 16, 16)] = g_v[
                        slot, 2 * r + 1, pl.ds(l * 16, 16)
                    ]
            return 0

        lax.fori_loop(0, PROWS // 4, rows4, 0)

    def mask(slot):
        zeros = jnp.zeros((16,), jnp.float32)
        for l in range(PAD_DIM // 16):
            p_v[slot, 0, pl.ds(l * 16, 16)] = zeros
        for l in range(EMBED_DIM // 16):
            p_v[slot, 1, pl.ds(l * 16, 16)] = zeros

    fire_idx(0, 0)
    fire_idx(1, 1)
    wait_idx(0)
    fire_gather(0, 0)
    wait_idx(1)
    fire_gather(1, 1)

    wait_gather(0)
    fire_idx(2, 0)
    repack(0)
    mask(0)
    fire_scatter(0, 0)
    wait_idx(0)
    fire_gather(2, 0)

    wait_gather(1)
    fire_idx(3, 1)
    repack(1)
    fire_scatter(1, 1)
    wait_idx(1)
    fire_gather(3, 1)

    def group(g, _):
        for b in range(2):
            c = 2 * g + 2 + b
            wait_gather(b)

            @pl.when(c + 2 < N_CHUNKS)
            def _():
                fire_idx(c + 2, b)

            wait_scatter(b)
            repack(b)
            if b == 0:
                mask(b)
            fire_scatter(c, b)

            @pl.when(c + 2 < N_CHUNKS)
            def _():
                wait_idx(b)
                fire_gather(c + 2, b)

        return 0

    lax.fori_loop(0, (N_CHUNKS - 2) // 2, group, 0)

    wait_scatter(0)
    wait_scatter(1)


@jax.jit
def _encode(kmer_indices, kmer_table):
    table_p = jnp.pad(kmer_table, ((0, 0), (0, PAD_DIM - EMBED_DIM)))
    idx_flat = kmer_indices.reshape(BATCH * SEQ_LEN)
    mesh = plsc.VectorSubcoreMesh(
        core_axis_name="c",
        subcore_axis_name="s",
        num_cores=NUM_CORES,
        num_subcores=NUM_SUBCORES,
    )
    run = pl.kernel(
        _sc_body,
        out_type=jax.ShapeDtypeStruct((OUT_ROWS, 128), jnp.float32),
        mesh=mesh,
        scratch_types=[
            pltpu.VMEM((CHUNK,), jnp.int32),
            pltpu.VMEM((CHUNK,), jnp.int32),
            pltpu.VMEM((2, CHUNK, PAD_DIM), jnp.float32),
            pltpu.VMEM((2, PROWS, PAD_DIM), jnp.float32),
            pltpu.SemaphoreType.DMA,
            pltpu.SemaphoreType.DMA,
            pltpu.SemaphoreType.DMA,
            pltpu.SemaphoreType.DMA,
            pltpu.SemaphoreType.DMA,
            pltpu.SemaphoreType.DMA,
        ],
    )
    out = run(idx_flat, table_p)
    return out.reshape(BATCH, SEQ_LEN, EMBED_DIM)


def kernel(kmer_indices, kmer_table):
    return _encode(kmer_indices, kmer_table)
